# Initial kernel scaffold; baseline (speedup 1.0000x reference)
import functools

import jax
import jax.numpy as jnp
from jax import lax
from jax.experimental import pallas as pl
from jax.experimental.pallas import tpu as pltpu
from jax.experimental.pallas import tpu_sc as plsc

N = 10000
C = 128
K = 27
P = 12000

NPAD = 10240
KC = K * C
KP = K * P
NW = 32
IDX_W = 128
ROWS_PER_W = 80
IDX_ROWS = NW * ROWS_PER_W
NB = 4
GROUPS = ROWS_PER_W // NB
TILE_ROWS = NPAD // 16
DUMMY_ROW = N

_MM_BM = 1024
_MM_BN = 1152


def _mm1_body(x_ref, w_ref, o_ref):
    o_ref[...] = jnp.dot(x_ref[...], w_ref[...],
                         preferred_element_type=jnp.float32)


def _mm1(x, w):
    grid = (NPAD // _MM_BM, KC // _MM_BN)
    return pl.pallas_call(
        _mm1_body,
        grid=grid,
        in_specs=[
            pl.BlockSpec((_MM_BM, C), lambda i, j: (i, 0)),
            pl.BlockSpec((C, _MM_BN), lambda i, j: (0, j)),
        ],
        out_specs=pl.BlockSpec((_MM_BM, _MM_BN), lambda i, j: (i, j)),
        out_shape=jax.ShapeDtypeStruct((NPAD, KC), jnp.float32),
    )(x, w)


def _mm2_body(a0_ref, a1_ref, b_ref, w_ref, o_ref):
    h = jnp.maximum(a0_ref[...] + a1_ref[...] + b_ref[...], 0.0)
    o_ref[...] = jnp.dot(h, w_ref[...], preferred_element_type=jnp.float32)


def _mm2(a0, a1, b, w):
    grid = (NPAD // _MM_BM, KC // _MM_BN)
    return pl.pallas_call(
        _mm2_body,
        grid=grid,
        in_specs=[
            pl.BlockSpec((_MM_BM, C), lambda i, j: (i, 0)),
            pl.BlockSpec((_MM_BM, C), lambda i, j: (i, 0)),
            pl.BlockSpec((1, C), lambda i, j: (0, 0)),
            pl.BlockSpec((C, _MM_BN), lambda i, j: (0, j)),
        ],
        out_specs=pl.BlockSpec((_MM_BM, _MM_BN), lambda i, j: (i, j)),
        out_shape=jax.ShapeDtypeStruct((NPAD, KC), jnp.float32),
    )(a0, a1, b, w)


def _final_body(a0_ref, a1_ref, b_ref, f_ref, o_ref):
    o_ref[...] = jnp.maximum(
        a0_ref[...] + a1_ref[...] + b_ref[...] + f_ref[...], 0.0)


def _final(a0, a1, b, f):
    grid = (NPAD // _MM_BM,)
    blk = pl.BlockSpec((_MM_BM, C), lambda i: (i, 0))
    return pl.pallas_call(
        _final_body,
        grid=grid,
        in_specs=[blk, blk, pl.BlockSpec((1, C), lambda i: (0, 0)), blk],
        out_specs=blk,
        out_shape=jax.ShapeDtypeStruct((NPAD, C), jnp.float32),
    )(a0, a1, b, f)


@functools.partial(
    pl.kernel,
    out_type=jax.ShapeDtypeStruct((2, NPAD, C), jnp.float32),
    mesh=plsc.VectorSubcoreMesh(core_axis_name="c", subcore_axis_name="s"),
    scratch_types=[
        pltpu.VMEM((NB, IDX_W), jnp.int32),
        pltpu.VMEM((NB, IDX_W), jnp.int32),
        pltpu.VMEM((NB, IDX_W, C), jnp.float32),
        pltpu.VMEM_SHARED((NPAD, C), jnp.float32),
        pltpu.SemaphoreType.DMA,
    ],
)
def _sc_gather_scatter(y_hbm, iin_hbm, iout_hbm, zeros_hbm, out_hbm,
                       iin_v, iout_v, rows_v, accum, sem):
    cid = lax.axis_index("c")
    sid = lax.axis_index("s")
    wid = sid * 2 + cid

    pltpu.sync_copy(zeros_hbm.at[pl.ds(sid * TILE_ROWS, TILE_ROWS)],
                    accum.at[pl.ds(sid * TILE_ROWS, TILE_ROWS)])
    plsc.subcore_barrier()

    base = wid * ROWS_PER_W

    def group(g, carry):
        r0 = base + g * NB
        pltpu.sync_copy(iin_hbm.at[pl.ds(r0, NB)], iin_v)
        pltpu.sync_copy(iout_hbm.at[pl.ds(r0, NB)], iout_v)
        cps = [pltpu.async_copy(y_hbm.at[iin_v.at[b]], rows_v.at[b], sem)
               for b in range(NB)]
        for cp in cps:
            cp.wait()
        for b in range(NB):
            pltpu.sync_copy(rows_v.at[b], accum.at[iout_v.at[b]], add=True)
        return carry

    lax.fori_loop(0, GROUPS, group, 0)

    plsc.subcore_barrier()
    pltpu.sync_copy(accum.at[pl.ds(sid * TILE_ROWS, TILE_ROWS)],
                    out_hbm.at[cid, pl.ds(sid * TILE_ROWS, TILE_ROWS)])


def kernel(feats, pairs_in, pairs_out, W1, g1, b1, m1, v1,
           W2, g2, b2, m2, v2):
    eps = 1e-5
    s1 = g1 * lax.rsqrt(v1 + eps)
    s2 = g2 * lax.rsqrt(v2 + eps)
    w1c = (W1 * s1[None, None, :]).transpose(1, 0, 2).reshape(C, KC)
    w2c = (W2 * s2[None, None, :]).transpose(1, 0, 2).reshape(C, KC)
    b1e = (b1 - m1 * s1).reshape(1, C)
    b2e = (b2 - m2 * s2).reshape(1, C)

    featsp = jnp.pad(feats, ((0, NPAD - N), (0, 0)))

    iin = (pairs_in * K + jnp.arange(K, dtype=jnp.int32)[:, None]).reshape(-1)
    iin = jnp.pad(iin, (0, IDX_ROWS * IDX_W - KP)).reshape(IDX_ROWS, IDX_W)
    iout = jnp.pad(pairs_out.reshape(-1), (0, IDX_ROWS * IDX_W - KP),
                   constant_values=DUMMY_ROW).reshape(IDX_ROWS, IDX_W)

    zeros = jnp.zeros((NPAD, C), dtype=jnp.float32)

    y1 = _mm1(featsp, w1c).reshape(NPAD * K, C)
    p1 = _sc_gather_scatter(y1, iin, iout, zeros)
    y2 = _mm2(p1[0], p1[1], b1e, w2c).reshape(NPAD * K, C)
    p2 = _sc_gather_scatter(y2, iin, iout, zeros)
    out = _final(p2[0], p2[1], b2e, featsp)
    return out[:N]


# speedup vs baseline: 3.7159x; 3.7159x over previous
import functools

import jax
import jax.numpy as jnp
from jax import lax
from jax.experimental import pallas as pl
from jax.experimental.pallas import tpu as pltpu
from jax.experimental.pallas import tpu_sc as plsc

N = 10000
C = 128
K = 27
P = 12000

NPAD = 10240
KC = K * C
KP = K * P
NW = 32
IDX_W = 128
ROWS_PER_W = 80
IDX_ROWS = NW * ROWS_PER_W
NB = 2
GROUPS = ROWS_PER_W // NB
TILE_ROWS = NPAD // 16
DUMMY_ROW = N

_MM_BM = 1024
_MM_BN = 1152


def _mm1_body(x_ref, w_ref, o_ref):
    o_ref[...] = jnp.dot(x_ref[...], w_ref[...],
                         preferred_element_type=jnp.float32)


def _mm1(x, w):
    grid = (NPAD // _MM_BM, KC // _MM_BN)
    return pl.pallas_call(
        _mm1_body,
        grid=grid,
        in_specs=[
            pl.BlockSpec((_MM_BM, C), lambda i, j: (i, 0)),
            pl.BlockSpec((C, _MM_BN), lambda i, j: (0, j)),
        ],
        out_specs=pl.BlockSpec((_MM_BM, _MM_BN), lambda i, j: (i, j)),
        out_shape=jax.ShapeDtypeStruct((NPAD, KC), jnp.float32),
    )(x, w)


def _mm2_body(a0_ref, a1_ref, b_ref, w_ref, o_ref):
    h = jnp.maximum(a0_ref[...] + a1_ref[...] + b_ref[...], 0.0)
    o_ref[...] = jnp.dot(h, w_ref[...], preferred_element_type=jnp.float32)


def _mm2(a0, a1, b, w):
    grid = (NPAD // _MM_BM, KC // _MM_BN)
    return pl.pallas_call(
        _mm2_body,
        grid=grid,
        in_specs=[
            pl.BlockSpec((_MM_BM, C), lambda i, j: (i, 0)),
            pl.BlockSpec((_MM_BM, C), lambda i, j: (i, 0)),
            pl.BlockSpec((1, C), lambda i, j: (0, 0)),
            pl.BlockSpec((C, _MM_BN), lambda i, j: (0, j)),
        ],
        out_specs=pl.BlockSpec((_MM_BM, _MM_BN), lambda i, j: (i, j)),
        out_shape=jax.ShapeDtypeStruct((NPAD, KC), jnp.float32),
    )(a0, a1, b, w)


def _final_body(a0_ref, a1_ref, b_ref, f_ref, o_ref):
    o_ref[...] = jnp.maximum(
        a0_ref[...] + a1_ref[...] + b_ref[...] + f_ref[...], 0.0)


def _final(a0, a1, b, f):
    grid = (NPAD // _MM_BM,)
    blk = pl.BlockSpec((_MM_BM, C), lambda i: (i, 0))
    return pl.pallas_call(
        _final_body,
        grid=grid,
        in_specs=[blk, blk, pl.BlockSpec((1, C), lambda i: (0, 0)), blk],
        out_specs=blk,
        out_shape=jax.ShapeDtypeStruct((NPAD, C), jnp.float32),
    )(a0, a1, b, f)


@functools.partial(
    pl.kernel,
    out_type=jax.ShapeDtypeStruct((2, NPAD, C), jnp.float32),
    mesh=plsc.VectorSubcoreMesh(core_axis_name="c", subcore_axis_name="s"),
    scratch_types=[
        pltpu.VMEM((NB, IDX_W), jnp.int32),
        pltpu.VMEM((NB, IDX_W), jnp.int32),
        pltpu.VMEM((NB, IDX_W, C), jnp.float32),
        pltpu.VMEM_SHARED((NPAD, C), jnp.float32),
        pltpu.SemaphoreType.DMA,
    ],
)
def _sc_gather_scatter(y_hbm, iin_hbm, iout_hbm, zeros_hbm, out_hbm,
                       iin_v, iout_v, rows_v, accum, sem):
    cid = lax.axis_index("c")
    sid = lax.axis_index("s")
    wid = sid * 2 + cid

    pltpu.sync_copy(zeros_hbm.at[pl.ds(sid * TILE_ROWS, TILE_ROWS)],
                    accum.at[pl.ds(sid * TILE_ROWS, TILE_ROWS)])
    plsc.subcore_barrier()

    base = wid * ROWS_PER_W

    def group(g, carry):
        r0 = base + g * NB
        pltpu.sync_copy(iin_hbm.at[pl.ds(r0, NB)], iin_v)
        pltpu.sync_copy(iout_hbm.at[pl.ds(r0, NB)], iout_v)
        cps = [pltpu.async_copy(y_hbm.at[iin_v.at[b]], rows_v.at[b], sem)
               for b in range(NB)]
        for cp in cps:
            cp.wait()
        for b in range(NB):
            pltpu.sync_copy(rows_v.at[b], accum.at[iout_v.at[b]], add=True)
        return carry

    lax.fori_loop(0, GROUPS, group, 0)

    plsc.subcore_barrier()
    pltpu.sync_copy(accum.at[pl.ds(sid * TILE_ROWS, TILE_ROWS)],
                    out_hbm.at[cid, pl.ds(sid * TILE_ROWS, TILE_ROWS)])


def kernel(feats, pairs_in, pairs_out, W1, g1, b1, m1, v1,
           W2, g2, b2, m2, v2):
    eps = 1e-5
    s1 = g1 * lax.rsqrt(v1 + eps)
    s2 = g2 * lax.rsqrt(v2 + eps)
    w1c = (W1 * s1[None, None, :]).transpose(1, 0, 2).reshape(C, KC)
    w2c = (W2 * s2[None, None, :]).transpose(1, 0, 2).reshape(C, KC)
    b1e = (b1 - m1 * s1).reshape(1, C)
    b2e = (b2 - m2 * s2).reshape(1, C)

    featsp = jnp.pad(feats, ((0, NPAD - N), (0, 0)))

    iin = (pairs_in * K + jnp.arange(K, dtype=jnp.int32)[:, None]).reshape(-1)
    iin = jnp.pad(iin, (0, IDX_ROWS * IDX_W - KP)).reshape(IDX_ROWS, IDX_W)
    iout = jnp.pad(pairs_out.reshape(-1), (0, IDX_ROWS * IDX_W - KP),
                   constant_values=DUMMY_ROW).reshape(IDX_ROWS, IDX_W)

    zeros = jnp.zeros((NPAD, C), dtype=jnp.float32)

    y1 = _mm1(featsp, w1c).reshape(NPAD * K, C)
    p1 = _sc_gather_scatter(y1, iin, iout, zeros)
    y2 = _mm2(p1[0], p1[1], b1e, w2c).reshape(NPAD * K, C)
    p2 = _sc_gather_scatter(y2, iin, iout, zeros)
    out = _final(p2[0], p2[1], b2e, featsp)
    return out[:N]
